# baseline (speedup 1.0000x reference)
import functools

import jax
import jax.numpy as jnp
from jax import lax
from jax.experimental import pallas as pl
from jax.experimental.pallas import tpu as pltpu
from jax.experimental.pallas import tpu_sc as plsc

NC = 2
NS = 16
B = 16384
D = 16
NT = 26
NB = 4
BH = B // NC


def _body(*refs):
    cat_refs = refs[0:NT]
    tab_refs = refs[NT:2 * NT]
    out_refs = refs[2 * NT:3 * NT]
    idx_bufs = refs[3 * NT:3 * NT + NB]
    col_bufs = refs[3 * NT + NB:3 * NT + 2 * NB]
    sem_i = refs[3 * NT + 2 * NB:3 * NT + 3 * NB]
    sem_g = refs[3 * NT + 3 * NB:3 * NT + 4 * NB]
    sem_s = refs[3 * NT + 4 * NB:3 * NT + 5 * NB]

    c = lax.axis_index("c")
    s = lax.axis_index("s")
    base = c * BH

    def idx_copy(t):
        b = t % NB
        return pltpu.async_copy(
            cat_refs[t].at[pl.ds(base, BH)], idx_bufs[b], sem_i[b])

    def gather(t):
        b = t % NB
        return pltpu.async_copy(
            tab_refs[t].at[s].at[idx_bufs[b]], col_bufs[b], sem_g[b])

    def store(t):
        b = t % NB
        return pltpu.async_copy(
            col_bufs[b], out_refs[t].at[s, pl.ds(base, BH)], sem_s[b])

    di = {}
    dg = {}
    ds = {}
    for step in range(NT + 2):
        ta, tb, tc = step, step - 1, step - 2
        if ta < NT:
            if ta - NB >= 0:
                ds[ta - NB].wait()
            di[ta] = idx_copy(ta)
        if 0 <= tb < NT:
            di[tb].wait()
            dg[tb] = gather(tb)
        if 0 <= tc < NT:
            dg[tc].wait()
            ds[tc] = store(tc)
    for t in range(max(0, NT - NB), NT):
        ds[t].wait()


@jax.jit
def _embed_all(cats, tables_t):
    mesh = plsc.VectorSubcoreMesh(
        core_axis_name="c", subcore_axis_name="s",
        num_cores=NC, num_subcores=NS,
    )
    out_type = tuple(
        jax.ShapeDtypeStruct((D, B), jnp.float32) for _ in range(NT)
    )
    fn = pl.kernel(
        _body,
        out_type=out_type,
        mesh=mesh,
        scratch_types=(
            [pltpu.VMEM((BH,), jnp.int32) for _ in range(NB)]
            + [pltpu.VMEM((BH,), jnp.float32) for _ in range(NB)]
            + [pltpu.SemaphoreType.DMA for _ in range(3 * NB)]
        ),
        compiler_params=pltpu.CompilerParams(use_tc_tiling_on_sc=False),
    )
    outs_t = fn(*cats, *tables_t)
    return tuple(o.T for o in outs_t)


def kernel(cat_0, table_0, cat_1, table_1, cat_2, table_2, cat_3, table_3, cat_4, table_4, cat_5, table_5, cat_6, table_6, cat_7, table_7, cat_8, table_8, cat_9, table_9, cat_10, table_10, cat_11, table_11, cat_12, table_12, cat_13, table_13, cat_14, table_14, cat_15, table_15, cat_16, table_16, cat_17, table_17, cat_18, table_18, cat_19, table_19, cat_20, table_20, cat_21, table_21, cat_22, table_22, cat_23, table_23, cat_24, table_24, cat_25, table_25):
    args = locals()
    cats = tuple(args[f"cat_{i}"] for i in range(NT))
    tables_t = tuple(args[f"table_{i}"].T for i in range(NT))
    return _embed_all(cats, tables_t)


# speedup vs baseline: 5.1740x; 5.1740x over previous
import functools

import jax
import jax.numpy as jnp
from jax import lax
from jax.experimental import pallas as pl
from jax.experimental.pallas import tpu as pltpu
from jax.experimental.pallas import tpu_sc as plsc

NC = 2
NS = 16
NW = NC * NS
B = 16384
D = 16
NT = 26
NL = 10
NSM = NT - NL
BW = B // NW
NB = 8
NBR = 4
BLK = 8192
VOCABS = (1000000, 1000000) + (100000,) * 8 + (1000,) * 16
NTASK = NL * D


def _flatten_class(tabs_t, g):
    n = len(tabs_t)
    v = tabs_t[0].shape[1]
    nb = -(-v // BLK)
    vp = nb * BLK

    def body(*refs):
        ins = refs[:n]
        outs = refs[n:]
        for tb in range(n):
            x = ins[tb][...]
            for k in range(8):
                outs[tb * 8 + k][...] = x[k]

    return pl.pallas_call(
        body,
        grid=(nb,),
        in_specs=[
            pl.BlockSpec((8, BLK), lambda i, g=g: (g, i)) for _ in range(n)
        ],
        out_specs=[
            pl.BlockSpec((BLK,), lambda i: (i,)) for _ in range(8 * n)
        ],
        out_shape=[
            jax.ShapeDtypeStruct((vp,), jnp.float32) for _ in range(8 * n)
        ],
    )(*tabs_t)


def _gather_body(*refs):
    cat_refs = refs[0:NT]
    big_refs = refs[NT:NT + NTASK]
    small_refs = refs[NT + NTASK:NT + NTASK + NSM]
    r = NT + NTASK + NSM
    bout_refs = refs[r:r + NL]
    sout_refs = refs[r + NL:r + NT]
    r += NT
    idx_all = refs[r]
    col_bufs = refs[r + 1:r + 1 + NB]
    sem_i = refs[r + 1 + NB]
    sem_g = refs[r + 2 + NB:r + 2 + 2 * NB]
    sem_s = refs[r + 2 + 2 * NB:r + 2 + 3 * NB]
    r += 2 + 3 * NB
    idx2 = refs[r]
    rows = refs[r + 1:r + 1 + NBR]
    sem_i2 = refs[r + 1 + NBR]
    sem_g2 = refs[r + 2 + NBR:r + 2 + 2 * NBR]
    sem_s2 = refs[r + 2 + 2 * NBR:r + 2 + 3 * NBR]

    c = lax.axis_index("c")
    s = lax.axis_index("s")
    wid = s * NC + c
    base = wid * BW

    ic1 = [
        pltpu.async_copy(
            cat_refs[t].at[pl.ds(base, BW)], idx_all.at[t], sem_i)
        for t in range(NL)
    ]
    ic2 = [
        pltpu.async_copy(
            cat_refs[NL + k].at[pl.ds(base, BW)], idx2.at[k], sem_i2)
        for k in range(NSM)
    ]
    for cp in ic1:
        cp.wait()

    def gather(j):
        t, k = divmod(j, D)
        return pltpu.async_copy(
            big_refs[j].at[idx_all.at[t]], col_bufs[j % NB], sem_g[j % NB])

    def store(j):
        t, k = divmod(j, D)
        return pltpu.async_copy(
            col_bufs[j % NB], bout_refs[t].at[k, pl.ds(base, BW)],
            sem_s[j % NB])

    dg = {}
    dst = {}
    for step in range(NTASK + 1):
        ja, jb = step, step - 1
        if ja < NTASK:
            if ja - NB >= 0:
                dst[ja - NB].wait()
            dg[ja] = gather(ja)
        if 0 <= jb < NTASK:
            dg[jb].wait()
            dst[jb] = store(jb)

    for cp in ic2:
        cp.wait()

    def rgather(k):
        return pltpu.async_copy(
            small_refs[k].at[idx2.at[k]], rows[k % NBR], sem_g2[k % NBR])

    def rstore(k):
        return pltpu.async_copy(
            rows[k % NBR], sout_refs[k].at[pl.ds(base, BW)], sem_s2[k % NBR])

    rg = {}
    rs = {}
    for step in range(NSM + 1):
        ka, kb = step, step - 1
        if ka < NSM:
            if ka - NBR >= 0:
                rs[ka - NBR].wait()
            rg[ka] = rgather(ka)
        if 0 <= kb < NSM:
            rg[kb].wait()
            rs[kb] = rstore(kb)
    for k in range(max(0, NSM - NBR), NSM):
        rs[k].wait()
    for j in range(max(0, NTASK - NB), NTASK):
        dst[j].wait()


@jax.jit
def _embed_all(cats, tables_t, smalls):
    big_lo = _flatten_class(tables_t[0:2], 0)
    big_hi = _flatten_class(tables_t[0:2], 1)
    mid_lo = _flatten_class(tables_t[2:NL], 0)
    mid_hi = _flatten_class(tables_t[2:NL], 1)
    flats = []
    for t in range(2):
        flats += list(big_lo[t * 8:t * 8 + 8]) + list(big_hi[t * 8:t * 8 + 8])
    for t in range(8):
        flats += list(mid_lo[t * 8:t * 8 + 8]) + list(mid_hi[t * 8:t * 8 + 8])

    mesh = plsc.VectorSubcoreMesh(
        core_axis_name="c", subcore_axis_name="s",
        num_cores=NC, num_subcores=NS,
    )
    out_type = tuple(
        [jax.ShapeDtypeStruct((D, B), jnp.float32) for _ in range(NL)]
        + [jax.ShapeDtypeStruct((B, D), jnp.float32) for _ in range(NSM)]
    )
    fn = pl.kernel(
        _gather_body,
        out_type=out_type,
        mesh=mesh,
        scratch_types=(
            [pltpu.VMEM((NL, BW), jnp.int32)]
            + [pltpu.VMEM((BW,), jnp.float32) for _ in range(NB)]
            + [pltpu.SemaphoreType.DMA for _ in range(1 + 2 * NB)]
            + [pltpu.VMEM((NSM, BW), jnp.int32)]
            + [pltpu.VMEM((BW, D), jnp.float32) for _ in range(NBR)]
            + [pltpu.SemaphoreType.DMA for _ in range(1 + 2 * NBR)]
        ),
        compiler_params=pltpu.CompilerParams(use_tc_tiling_on_sc=False),
    )
    outs = fn(*cats, *flats, *smalls)
    return tuple(
        [o.T for o in outs[:NL]] + list(outs[NL:])
    )


def kernel(cat_0, table_0, cat_1, table_1, cat_2, table_2, cat_3, table_3, cat_4, table_4, cat_5, table_5, cat_6, table_6, cat_7, table_7, cat_8, table_8, cat_9, table_9, cat_10, table_10, cat_11, table_11, cat_12, table_12, cat_13, table_13, cat_14, table_14, cat_15, table_15, cat_16, table_16, cat_17, table_17, cat_18, table_18, cat_19, table_19, cat_20, table_20, cat_21, table_21, cat_22, table_22, cat_23, table_23, cat_24, table_24, cat_25, table_25):
    args = locals()
    cats = tuple(args[f"cat_{i}"] for i in range(NT))
    tables_t = tuple(args[f"table_{i}"].T for i in range(NL))
    smalls = tuple(args[f"table_{i}"] for i in range(NL, NT))
    return _embed_all(cats, tables_t, smalls)


# speedup vs baseline: 5.6946x; 1.1006x over previous
import functools

import jax
import jax.numpy as jnp
from jax import lax
from jax.experimental import pallas as pl
from jax.experimental.pallas import tpu as pltpu
from jax.experimental.pallas import tpu_sc as plsc

NC = 2
NS = 16
NW = NC * NS
B = 16384
D = 16
NT = 26
NL = 10
NSM = NT - NL
BW = B // NW
NB = 8
NBR = 4
BLK = 8192
VOCABS = (1000000, 1000000) + (100000,) * 8 + (1000,) * 16
NTASK = NL * D


def _flatten_class(tabs_t):
    n = len(tabs_t)
    v = tabs_t[0].shape[1]
    nb = -(-v // BLK)
    vp = nb * BLK

    def body(*refs):
        ins = refs[:n]
        outs = refs[n:]
        for tb in range(n):
            x = ins[tb][...]
            for k in range(D):
                outs[tb * D + k][...] = x[k]

    return pl.pallas_call(
        body,
        grid=(nb,),
        in_specs=[
            pl.BlockSpec((D, BLK), lambda i: (0, i)) for _ in range(n)
        ],
        out_specs=[
            pl.BlockSpec((BLK,), lambda i: (i,)) for _ in range(D * n)
        ],
        out_shape=[
            jax.ShapeDtypeStruct((vp,), jnp.float32) for _ in range(D * n)
        ],
    )(*tabs_t)


def _gather_body(*refs):
    cat_refs = refs[0:NT]
    big_refs = refs[NT:NT + NTASK]
    small_refs = refs[NT + NTASK:NT + NTASK + NSM]
    r = NT + NTASK + NSM
    bout_refs = refs[r:r + NL]
    sout_refs = refs[r + NL:r + NT]
    r += NT
    idx_all = refs[r]
    col_bufs = refs[r + 1:r + 1 + NB]
    sem_i = refs[r + 1 + NB]
    sem_g = refs[r + 2 + NB:r + 2 + 2 * NB]
    sem_s = refs[r + 2 + 2 * NB:r + 2 + 3 * NB]
    r += 2 + 3 * NB
    idx2 = refs[r]
    rows = refs[r + 1:r + 1 + NBR]
    sem_i2 = refs[r + 1 + NBR]
    sem_g2 = refs[r + 2 + NBR:r + 2 + 2 * NBR]
    sem_s2 = refs[r + 2 + 2 * NBR:r + 2 + 3 * NBR]

    c = lax.axis_index("c")
    s = lax.axis_index("s")
    wid = s * NC + c
    base = wid * BW

    ic1 = [
        pltpu.async_copy(
            cat_refs[t].at[pl.ds(base, BW)], idx_all.at[t], sem_i)
        for t in range(NL)
    ]
    ic2 = [
        pltpu.async_copy(
            cat_refs[NL + k].at[pl.ds(base, BW)], idx2.at[k], sem_i2)
        for k in range(NSM)
    ]
    for cp in ic1:
        cp.wait()

    def gather(j):
        t, k = divmod(j, D)
        return pltpu.async_copy(
            big_refs[j].at[idx_all.at[t]], col_bufs[j % NB], sem_g[j % NB])

    def store(j):
        t, k = divmod(j, D)
        return pltpu.async_copy(
            col_bufs[j % NB], bout_refs[t].at[k, pl.ds(base, BW)],
            sem_s[j % NB])

    dg = {}
    dst = {}
    for step in range(NTASK + 1):
        ja, jb = step, step - 1
        if ja < NTASK:
            if ja - NB >= 0:
                dst[ja - NB].wait()
            dg[ja] = gather(ja)
        if 0 <= jb < NTASK:
            dg[jb].wait()
            dst[jb] = store(jb)

    for cp in ic2:
        cp.wait()

    def rgather(k):
        return pltpu.async_copy(
            small_refs[k].at[idx2.at[k]], rows[k % NBR], sem_g2[k % NBR])

    def rstore(k):
        return pltpu.async_copy(
            rows[k % NBR], sout_refs[k].at[pl.ds(base, BW)], sem_s2[k % NBR])

    rg = {}
    rs = {}
    for step in range(NSM + 1):
        ka, kb = step, step - 1
        if ka < NSM:
            if ka - NBR >= 0:
                rs[ka - NBR].wait()
            rg[ka] = rgather(ka)
        if 0 <= kb < NSM:
            rg[kb].wait()
            rs[kb] = rstore(kb)
    for k in range(max(0, NSM - NBR), NSM):
        rs[k].wait()
    for j in range(max(0, NTASK - NB), NTASK):
        dst[j].wait()


@jax.jit
def _embed_all(cats, tables_t, smalls):
    flats = list(_flatten_class(tables_t[0:2])) + list(
        _flatten_class(tables_t[2:NL]))

    mesh = plsc.VectorSubcoreMesh(
        core_axis_name="c", subcore_axis_name="s",
        num_cores=NC, num_subcores=NS,
    )
    out_type = tuple(
        [jax.ShapeDtypeStruct((D, B), jnp.float32) for _ in range(NL)]
        + [jax.ShapeDtypeStruct((B, D), jnp.float32) for _ in range(NSM)]
    )
    fn = pl.kernel(
        _gather_body,
        out_type=out_type,
        mesh=mesh,
        scratch_types=(
            [pltpu.VMEM((NL, BW), jnp.int32)]
            + [pltpu.VMEM((BW,), jnp.float32) for _ in range(NB)]
            + [pltpu.SemaphoreType.DMA for _ in range(1 + 2 * NB)]
            + [pltpu.VMEM((NSM, BW), jnp.int32)]
            + [pltpu.VMEM((BW, D), jnp.float32) for _ in range(NBR)]
            + [pltpu.SemaphoreType.DMA for _ in range(1 + 2 * NBR)]
        ),
        compiler_params=pltpu.CompilerParams(use_tc_tiling_on_sc=False),
    )
    outs = fn(*cats, *flats, *smalls)
    return tuple(
        [o.T for o in outs[:NL]] + list(outs[NL:])
    )


def kernel(cat_0, table_0, cat_1, table_1, cat_2, table_2, cat_3, table_3, cat_4, table_4, cat_5, table_5, cat_6, table_6, cat_7, table_7, cat_8, table_8, cat_9, table_9, cat_10, table_10, cat_11, table_11, cat_12, table_12, cat_13, table_13, cat_14, table_14, cat_15, table_15, cat_16, table_16, cat_17, table_17, cat_18, table_18, cat_19, table_19, cat_20, table_20, cat_21, table_21, cat_22, table_22, cat_23, table_23, cat_24, table_24, cat_25, table_25):
    args = locals()
    cats = tuple(args[f"cat_{i}"] for i in range(NT))
    tables_t = tuple(args[f"table_{i}"].T for i in range(NL))
    smalls = tuple(args[f"table_{i}"] for i in range(NL, NT))
    return _embed_all(cats, tables_t, smalls)


# speedup vs baseline: 6.2731x; 1.1016x over previous
import functools

import jax
import jax.numpy as jnp
from jax import lax
from jax.experimental import pallas as pl
from jax.experimental.pallas import tpu as pltpu
from jax.experimental.pallas import tpu_sc as plsc

NC = 2
NS = 16
NW = NC * NS
B = 16384
D = 16
NT = 26
NL = 10
NSM = NT - NL
BW = B // NW
NB = 8
NBR = 4
BLK_BIG = 65536
BLK_MID = 16384
VOCABS = (1000000, 1000000) + (100000,) * 8 + (1000,) * 16
NTASK = NL * D


def _flatten_class(tabs_t, blk):
    n = len(tabs_t)
    v = tabs_t[0].shape[1]
    nb = -(-v // blk)
    vp = nb * blk

    def body(*refs):
        ins = refs[:n]
        outs = refs[n:]
        for tb in range(n):
            x = ins[tb][...]
            for k in range(D):
                outs[tb * D + k][...] = x[k]

    return pl.pallas_call(
        body,
        grid=(nb,),
        in_specs=[
            pl.BlockSpec((D, blk), lambda i: (0, i)) for _ in range(n)
        ],
        out_specs=[
            pl.BlockSpec((blk,), lambda i: (i,)) for _ in range(D * n)
        ],
        out_shape=[
            jax.ShapeDtypeStruct((vp,), jnp.float32) for _ in range(D * n)
        ],
    )(*tabs_t)


def _gather_body(*refs):
    cat_refs = refs[0:NT]
    big_refs = refs[NT:NT + NTASK]
    small_refs = refs[NT + NTASK:NT + NTASK + NSM]
    r = NT + NTASK + NSM
    bout_refs = refs[r:r + NL]
    sout_refs = refs[r + NL:r + NT]
    r += NT
    idx_all = refs[r]
    col_bufs = refs[r + 1:r + 1 + NB]
    sem_i = refs[r + 1 + NB]
    sem_g = refs[r + 2 + NB:r + 2 + 2 * NB]
    sem_s = refs[r + 2 + 2 * NB:r + 2 + 3 * NB]
    r += 2 + 3 * NB
    idx2 = refs[r]
    rows = refs[r + 1:r + 1 + NBR]
    sem_i2 = refs[r + 1 + NBR]
    sem_g2 = refs[r + 2 + NBR:r + 2 + 2 * NBR]
    sem_s2 = refs[r + 2 + 2 * NBR:r + 2 + 3 * NBR]

    c = lax.axis_index("c")
    s = lax.axis_index("s")
    wid = s * NC + c
    base = wid * BW

    ic1 = [
        pltpu.async_copy(
            cat_refs[t].at[pl.ds(base, BW)], idx_all.at[t], sem_i)
        for t in range(NL)
    ]
    ic2 = [
        pltpu.async_copy(
            cat_refs[NL + k].at[pl.ds(base, BW)], idx2.at[k], sem_i2)
        for k in range(NSM)
    ]
    for cp in ic1:
        cp.wait()

    def gather(j):
        t, k = divmod(j, D)
        return pltpu.async_copy(
            big_refs[j].at[idx_all.at[t]], col_bufs[j % NB], sem_g[j % NB])

    def store(j):
        t, k = divmod(j, D)
        return pltpu.async_copy(
            col_bufs[j % NB], bout_refs[t].at[k, pl.ds(base, BW)],
            sem_s[j % NB])

    dg = {}
    dst = {}
    for step in range(NTASK + 1):
        ja, jb = step, step - 1
        if ja < NTASK:
            if ja - NB >= 0:
                dst[ja - NB].wait()
            dg[ja] = gather(ja)
        if 0 <= jb < NTASK:
            dg[jb].wait()
            dst[jb] = store(jb)

    for cp in ic2:
        cp.wait()

    def rgather(k):
        return pltpu.async_copy(
            small_refs[k].at[idx2.at[k]], rows[k % NBR], sem_g2[k % NBR])

    def rstore(k):
        return pltpu.async_copy(
            rows[k % NBR], sout_refs[k].at[pl.ds(base, BW)], sem_s2[k % NBR])

    rg = {}
    rs = {}
    for step in range(NSM + 1):
        ka, kb = step, step - 1
        if ka < NSM:
            if ka - NBR >= 0:
                rs[ka - NBR].wait()
            rg[ka] = rgather(ka)
        if 0 <= kb < NSM:
            rg[kb].wait()
            rs[kb] = rstore(kb)
    for k in range(max(0, NSM - NBR), NSM):
        rs[k].wait()
    for j in range(max(0, NTASK - NB), NTASK):
        dst[j].wait()


@jax.jit
def _embed_all(cats, tables_t, smalls):
    flats = list(_flatten_class(tables_t[0:2], BLK_BIG)) + list(
        _flatten_class(tables_t[2:NL], BLK_MID))

    mesh = plsc.VectorSubcoreMesh(
        core_axis_name="c", subcore_axis_name="s",
        num_cores=NC, num_subcores=NS,
    )
    out_type = tuple(
        [jax.ShapeDtypeStruct((D, B), jnp.float32) for _ in range(NL)]
        + [jax.ShapeDtypeStruct((B, D), jnp.float32) for _ in range(NSM)]
    )
    fn = pl.kernel(
        _gather_body,
        out_type=out_type,
        mesh=mesh,
        scratch_types=(
            [pltpu.VMEM((NL, BW), jnp.int32)]
            + [pltpu.VMEM((BW,), jnp.float32) for _ in range(NB)]
            + [pltpu.SemaphoreType.DMA for _ in range(1 + 2 * NB)]
            + [pltpu.VMEM((NSM, BW), jnp.int32)]
            + [pltpu.VMEM((BW, D), jnp.float32) for _ in range(NBR)]
            + [pltpu.SemaphoreType.DMA for _ in range(1 + 2 * NBR)]
        ),
        compiler_params=pltpu.CompilerParams(use_tc_tiling_on_sc=False),
    )
    outs = fn(*cats, *flats, *smalls)
    return tuple(
        [o.T for o in outs[:NL]] + list(outs[NL:])
    )


def kernel(cat_0, table_0, cat_1, table_1, cat_2, table_2, cat_3, table_3, cat_4, table_4, cat_5, table_5, cat_6, table_6, cat_7, table_7, cat_8, table_8, cat_9, table_9, cat_10, table_10, cat_11, table_11, cat_12, table_12, cat_13, table_13, cat_14, table_14, cat_15, table_15, cat_16, table_16, cat_17, table_17, cat_18, table_18, cat_19, table_19, cat_20, table_20, cat_21, table_21, cat_22, table_22, cat_23, table_23, cat_24, table_24, cat_25, table_25):
    args = locals()
    cats = tuple(args[f"cat_{i}"] for i in range(NT))
    tables_t = tuple(args[f"table_{i}"].T for i in range(NL))
    smalls = tuple(args[f"table_{i}"] for i in range(NL, NT))
    return _embed_all(cats, tables_t, smalls)
